# baseline (speedup 1.0000x reference)
import functools

import jax
import jax.numpy as jnp
from jax import lax
from jax.experimental import pallas as pl
from jax.experimental.pallas import tpu as pltpu
from jax.experimental.pallas import tpu_sc as plsc

M, F, K = 4, 4096, 12
CIN, HID = 256, 128
MF = M * F
FB = 512
NMB = F // FB
GRID = MF // FB
NW = 32
RPW = MF // NW
SPW = RPW // 8
GW = 13 * 8
NV = HID // 16
N_TOT = float(MF)
EPS = 1e-5

_mesh = plsc.VectorSubcoreMesh(core_axis_name="c", subcore_axis_name="s")


@functools.partial(
    pl.kernel,
    out_type=[jax.ShapeDtypeStruct((MF, HID), jnp.float32),
              jax.ShapeDtypeStruct((2 * NW, HID), jnp.float32)],
    mesh=_mesh,
    scratch_types=[
        pltpu.VMEM((SPW, GW), jnp.int32),
        [pltpu.VMEM((GW, HID), jnp.float32)] * 4,
        [pltpu.VMEM((8, HID), jnp.float32)] * 4,
        pltpu.VMEM((2, HID), jnp.float32),
        [pltpu.SemaphoreType.DMA] * 4,
        [pltpu.SemaphoreType.DMA] * 4,
    ],
)
def _gsum(z_hbm, idx_hbm, out_hbm, st_hbm, idx_v, rows, outv, st_v, sg, so):
    wid = lax.axis_index("s") * 2 + lax.axis_index("c")
    pltpu.sync_copy(idx_hbm.at[wid], idx_v)

    zero = jnp.zeros((16,), jnp.float32)
    for v in range(NV):
        st_v[0, pl.ds(v * 16, 16)] = zero
        st_v[1, pl.ds(v * 16, 16)] = zero

    for b in range(4):
        pltpu.async_copy(z_hbm.at[idx_v.at[b]], rows[b], sg[b])

    def body(i, carry):
        for b in range(4):
            s = 4 * i + b
            pltpu.make_async_copy(z_hbm.at[idx_v.at[s]], rows[b], sg[b]).wait()

            @pl.when(i > 0)
            def _drain():
                pltpu.make_async_copy(
                    outv[b], out_hbm.at[pl.ds(wid * RPW + (s - 4) * 8, 8)],
                    so[b]).wait()

            for v in range(NV):
                sl = pl.ds(v * 16, 16)
                faces = []
                for r in range(8):
                    acc = rows[b][r * 13, sl]
                    for j in range(1, 13):
                        acc = acc + rows[b][r * 13 + j, sl]
                    outv[b][r, sl] = acc
                    faces.append(acc)
                def _tree(xs):
                    while len(xs) > 1:
                        xs = [a + c for a, c in zip(xs[::2], xs[1::2])]
                    return xs[0]
                plsc.addupdate(st_v.at[0, sl], _tree(faces))
                plsc.addupdate(st_v.at[1, sl], _tree([a * a for a in faces]))
            pltpu.async_copy(outv[b], out_hbm.at[pl.ds(wid * RPW + s * 8, 8)],
                             so[b])

            @pl.when(i < SPW // 4 - 1)
            def _prefetch():
                pltpu.async_copy(z_hbm.at[idx_v.at[s + 4]], rows[b], sg[b])
        return carry

    lax.fori_loop(0, SPW // 4, body, 0)

    for b in range(4):
        pltpu.make_async_copy(
            outv[b], out_hbm.at[pl.ds(wid * RPW + (SPW - 4 + b) * 8, 8)],
            so[b]).wait()
    pltpu.sync_copy(st_v.at[pl.ds(0, 1)], st_hbm.at[pl.ds(wid, 1)])
    pltpu.sync_copy(st_v.at[pl.ds(1, 1)], st_hbm.at[pl.ds(NW + wid, 1)])


def _mm0_body(fea_ref, w1_ref, wp_ref, z_ref, p_ref):
    x = fea_ref[0]
    dn = (((0,), (1,)), ((), ()))
    z_ref[...] = lax.dot_general(x, w1_ref[...], dn, preferred_element_type=jnp.float32)
    p_ref[...] = lax.dot_general(x, wp_ref[...], dn, preferred_element_type=jnp.float32)


def _bn_act(s_ref, st_ref, g_ref, be_ref):
    st = st_ref[...]
    mean = jnp.sum(st[0:NW], axis=0, keepdims=True) * (1.0 / N_TOT)
    var = jnp.sum(st[NW:], axis=0, keepdims=True) * (1.0 / N_TOT) - mean * mean
    scale = g_ref[...] * lax.rsqrt(var + EPS)
    return jnp.maximum((s_ref[...] - mean) * scale + be_ref[...], 0.0)


def _bnmm_body(s_ref, st_ref, g_ref, be_ref, w_ref, z_ref):
    a = _bn_act(s_ref, st_ref, g_ref, be_ref)
    z_ref[...] = lax.dot_general(a, w_ref[...], (((1,), (1,)), ((), ())),
                                 preferred_element_type=jnp.float32)


def _bnmm4_body(s_ref, st_ref, g_ref, be_ref, w_ref, p_ref, z_ref, h_ref):
    h = _bn_act(s_ref, st_ref, g_ref, be_ref)
    h_ref[...] = h
    z_ref[...] = p_ref[...] + lax.dot_general(h, w_ref[...], (((1,), (1,)), ((), ())),
                                              preferred_element_type=jnp.float32)


def _final_body(fea_ref, h0_ref, s_ref, st_ref, g_ref, be_ref, o_ref):
    h1 = _bn_act(s_ref, st_ref, g_ref, be_ref)
    o_ref[0] = jnp.concatenate([fea_ref[0], h0_ref[...].T, h1.T], axis=0)


_full = pl.BlockSpec((HID, HID), lambda i: (0, 0))
_row = pl.BlockSpec((1, HID), lambda i: (0, 0))
_st = pl.BlockSpec((2 * NW, HID), lambda i: (0, 0))
_sblk = pl.BlockSpec((FB, HID), lambda i: (i, 0))


def _bnmm(s, st, g, be, w):
    return pl.pallas_call(
        _bnmm_body,
        grid=(GRID,),
        in_specs=[_sblk, _st, _row, _row, _full],
        out_specs=_sblk,
        out_shape=jax.ShapeDtypeStruct((MF, HID), jnp.float32),
    )(s, st, g, be, w)


def kernel(fea, ring_n, W1_0, b1_0, g1_0, be1_0, W2_0, b2_0, g2_0, be2_0,
           W1_1, b1_1, g1_1, be1_1, W2_1, b2_1, g2_1, be2_1):
    ring = ring_n.astype(jnp.int32)
    base = (jnp.arange(M, dtype=jnp.int32) * F)[:, None, None]
    centers = base + jnp.arange(F, dtype=jnp.int32)[None, :, None]
    idx_all = jnp.concatenate([centers, ring + base], axis=2).reshape(NW, SPW, GW)

    g1_0r, be1_0r = g1_0.reshape(1, HID), be1_0.reshape(1, HID)
    g2_0r, be2_0r = g2_0.reshape(1, HID), be2_0.reshape(1, HID)
    g1_1r, be1_1r = g1_1.reshape(1, HID), be1_1.reshape(1, HID)
    g2_1r, be2_1r = g2_1.reshape(1, HID), be2_1.reshape(1, HID)
    Wp = W1_1[:, :CIN]
    Wh = W1_1[:, CIN:]

    z1, p = pl.pallas_call(
        _mm0_body,
        grid=(M, NMB),
        in_specs=[
            pl.BlockSpec((1, CIN, FB), lambda m, fb: (m, 0, fb)),
            pl.BlockSpec((HID, CIN), lambda m, fb: (0, 0)),
            pl.BlockSpec((HID, CIN), lambda m, fb: (0, 0)),
        ],
        out_specs=[
            pl.BlockSpec((FB, HID), lambda m, fb: (m * NMB + fb, 0)),
            pl.BlockSpec((FB, HID), lambda m, fb: (m * NMB + fb, 0)),
        ],
        out_shape=[jax.ShapeDtypeStruct((MF, HID), jnp.float32)] * 2,
    )(fea, W1_0, Wp)

    s1, st1 = _gsum(z1, idx_all)
    z2 = _bnmm(s1, st1, g1_0r, be1_0r, W2_0)
    s2, st2 = _gsum(z2, idx_all)
    z3, h0 = pl.pallas_call(
        _bnmm4_body,
        grid=(GRID,),
        in_specs=[_sblk, _st, _row, _row, _full, _sblk],
        out_specs=[_sblk, _sblk],
        out_shape=[jax.ShapeDtypeStruct((MF, HID), jnp.float32)] * 2,
    )(s2, st2, g2_0r, be2_0r, Wh, p)
    s3, st3 = _gsum(z3, idx_all)
    z4 = _bnmm(s3, st3, g1_1r, be1_1r, W2_1)
    s4, st4 = _gsum(z4, idx_all)
    out = pl.pallas_call(
        _final_body,
        grid=(M, NMB),
        in_specs=[
            pl.BlockSpec((1, CIN, FB), lambda m, fb: (m, 0, fb)),
            pl.BlockSpec((FB, HID), lambda m, fb: (m * NMB + fb, 0)),
            pl.BlockSpec((FB, HID), lambda m, fb: (m * NMB + fb, 0)),
            pl.BlockSpec((2 * NW, HID), lambda m, fb: (0, 0)),
            pl.BlockSpec((1, HID), lambda m, fb: (0, 0)),
            pl.BlockSpec((1, HID), lambda m, fb: (0, 0)),
        ],
        out_specs=pl.BlockSpec((1, CIN + 2 * HID, FB), lambda m, fb: (m, 0, fb)),
        out_shape=jax.ShapeDtypeStruct((M, CIN + 2 * HID, F), jnp.float32),
    )(fea, h0, s4, st4, g2_1r, be2_1r)
    return out


# speedup vs baseline: 1.0834x; 1.0834x over previous
import functools

import jax
import jax.numpy as jnp
from jax import lax
from jax.experimental import pallas as pl
from jax.experimental.pallas import tpu as pltpu
from jax.experimental.pallas import tpu_sc as plsc

M, F, K = 4, 4096, 12
CIN, HID = 256, 128
MF = M * F
FB = 512
NMB = F // FB
GRID = MF // FB
NW = 32
RPW = MF // NW
SPW = RPW // 8
GW = 13 * 8
NV = HID // 16
N_TOT = float(MF)
EPS = 1e-5

_mesh = plsc.VectorSubcoreMesh(core_axis_name="c", subcore_axis_name="s")


@functools.partial(
    pl.kernel,
    out_type=[jax.ShapeDtypeStruct((MF, HID), jnp.float32),
              jax.ShapeDtypeStruct((2 * NW, HID), jnp.float32)],
    mesh=_mesh,
    scratch_types=[
        pltpu.VMEM((SPW, GW), jnp.int32),
        pltpu.VMEM((GW, HID), jnp.float32),
        pltpu.VMEM((GW, HID), jnp.float32),
        pltpu.VMEM((8, HID), jnp.float32),
        pltpu.VMEM((8, HID), jnp.float32),
        pltpu.VMEM((2, HID), jnp.float32),
        pltpu.SemaphoreType.DMA,
        pltpu.SemaphoreType.DMA,
        pltpu.SemaphoreType.DMA,
        pltpu.SemaphoreType.DMA,
    ],
)
def _gsum(z_hbm, idx_hbm, out_hbm, st_hbm, idx_v, rows0, rows1, outv0, outv1,
          st_v, sg0, sg1, so0, so1):
    wid = lax.axis_index("s") * 2 + lax.axis_index("c")
    pltpu.sync_copy(idx_hbm.at[wid], idx_v)
    rows = (rows0, rows1)
    outv = (outv0, outv1)
    sg = (sg0, sg1)
    so = (so0, so1)

    zero = jnp.zeros((16,), jnp.float32)
    for v in range(NV):
        st_v[0, pl.ds(v * 16, 16)] = zero
        st_v[1, pl.ds(v * 16, 16)] = zero

    pltpu.async_copy(z_hbm.at[idx_v.at[0]], rows0, sg0)
    pltpu.async_copy(z_hbm.at[idx_v.at[1]], rows1, sg1)

    def body(i, carry):
        for b in range(2):
            s = 2 * i + b
            pltpu.make_async_copy(z_hbm.at[idx_v.at[s]], rows[b], sg[b]).wait()

            @pl.when(i > 0)
            def _drain():
                pltpu.make_async_copy(
                    outv[b], out_hbm.at[pl.ds(wid * RPW + (s - 2) * 8, 8)],
                    so[b]).wait()

            for v in range(NV):
                sl = pl.ds(v * 16, 16)
                faces = []
                for r in range(8):
                    acc = rows[b][r * 13, sl]
                    for j in range(1, 13):
                        acc = acc + rows[b][r * 13 + j, sl]
                    outv[b][r, sl] = acc
                    faces.append(acc)
                def _tree(xs):
                    while len(xs) > 1:
                        xs = [a + c for a, c in zip(xs[::2], xs[1::2])]
                    return xs[0]
                plsc.addupdate(st_v.at[0, sl], _tree(faces))
                plsc.addupdate(st_v.at[1, sl], _tree([a * a for a in faces]))
            pltpu.async_copy(outv[b], out_hbm.at[pl.ds(wid * RPW + s * 8, 8)],
                             so[b])

            @pl.when(i < SPW // 2 - 1)
            def _prefetch():
                pltpu.async_copy(z_hbm.at[idx_v.at[s + 2]], rows[b], sg[b])
        return carry

    lax.fori_loop(0, SPW // 2, body, 0)

    for b in range(2):
        pltpu.make_async_copy(
            outv[b], out_hbm.at[pl.ds(wid * RPW + (SPW - 2 + b) * 8, 8)],
            so[b]).wait()
    pltpu.sync_copy(st_v.at[pl.ds(0, 1)], st_hbm.at[pl.ds(wid, 1)])
    pltpu.sync_copy(st_v.at[pl.ds(1, 1)], st_hbm.at[pl.ds(NW + wid, 1)])


def _mm0_body(fea_ref, w1_ref, wp_ref, z_ref, p_ref):
    x = fea_ref[0]
    dn = (((0,), (1,)), ((), ()))
    z_ref[...] = lax.dot_general(x, w1_ref[...], dn, preferred_element_type=jnp.float32)
    p_ref[...] = lax.dot_general(x, wp_ref[...], dn,
                                 preferred_element_type=jnp.float32
                                 ).astype(jnp.bfloat16)


def _bn_act(s_ref, st_ref, g_ref, be_ref):
    st = st_ref[...]
    mean = jnp.sum(st[0:NW], axis=0, keepdims=True) * (1.0 / N_TOT)
    var = jnp.sum(st[NW:], axis=0, keepdims=True) * (1.0 / N_TOT) - mean * mean
    scale = g_ref[...] * lax.rsqrt(var + EPS)
    return jnp.maximum((s_ref[...] - mean) * scale + be_ref[...], 0.0)


def _bnmm_body(s_ref, st_ref, g_ref, be_ref, w_ref, z_ref):
    a = _bn_act(s_ref, st_ref, g_ref, be_ref)
    z_ref[...] = lax.dot_general(a, w_ref[...], (((1,), (1,)), ((), ())),
                                 preferred_element_type=jnp.float32)


def _bnmm4_body(s_ref, st_ref, g_ref, be_ref, w_ref, p_ref, z_ref, h_ref):
    h = _bn_act(s_ref, st_ref, g_ref, be_ref)
    h_ref[...] = h
    z_ref[...] = (p_ref[...].astype(jnp.float32)
                  + lax.dot_general(h, w_ref[...], (((1,), (1,)), ((), ())),
                                    preferred_element_type=jnp.float32))


def _copy_body(fea_ref, o_ref):
    o_ref[...] = fea_ref[...]


def _final_body(buf_ref, h0_ref, s_ref, st_ref, g_ref, be_ref, o_ref):
    del buf_ref
    h1 = _bn_act(s_ref, st_ref, g_ref, be_ref)
    o_ref[0] = jnp.concatenate([h0_ref[...].T, h1.T], axis=0)


_full = pl.BlockSpec((HID, HID), lambda i: (0, 0))
_row = pl.BlockSpec((1, HID), lambda i: (0, 0))
_st = pl.BlockSpec((2 * NW, HID), lambda i: (0, 0))
_sblk = pl.BlockSpec((FB, HID), lambda i: (i, 0))


def _bnmm(s, st, g, be, w):
    return pl.pallas_call(
        _bnmm_body,
        grid=(GRID,),
        in_specs=[_sblk, _st, _row, _row, _full],
        out_specs=_sblk,
        out_shape=jax.ShapeDtypeStruct((MF, HID), jnp.float32),
    )(s, st, g, be, w)


def kernel(fea, ring_n, W1_0, b1_0, g1_0, be1_0, W2_0, b2_0, g2_0, be2_0,
           W1_1, b1_1, g1_1, be1_1, W2_1, b2_1, g2_1, be2_1):
    ring = ring_n.astype(jnp.int32)
    base = (jnp.arange(M, dtype=jnp.int32) * F)[:, None, None]
    centers = base + jnp.arange(F, dtype=jnp.int32)[None, :, None]
    idx_all = jnp.concatenate([centers, ring + base], axis=2).reshape(NW, SPW, GW)

    g1_0r, be1_0r = g1_0.reshape(1, HID), be1_0.reshape(1, HID)
    g2_0r, be2_0r = g2_0.reshape(1, HID), be2_0.reshape(1, HID)
    g1_1r, be1_1r = g1_1.reshape(1, HID), be1_1.reshape(1, HID)
    g2_1r, be2_1r = g2_1.reshape(1, HID), be2_1.reshape(1, HID)
    Wp = W1_1[:, :CIN]
    Wh = W1_1[:, CIN:]

    out0 = pl.pallas_call(
        _copy_body,
        grid=(M, NMB),
        in_specs=[pl.BlockSpec((1, CIN, FB), lambda m, fb: (m, 0, fb))],
        out_specs=pl.BlockSpec((1, CIN, FB), lambda m, fb: (m, 0, fb)),
        out_shape=jax.ShapeDtypeStruct((M, CIN + 2 * HID, F), jnp.float32),
    )(fea)

    z1, p = pl.pallas_call(
        _mm0_body,
        grid=(M, NMB),
        in_specs=[
            pl.BlockSpec((1, CIN, FB), lambda m, fb: (m, 0, fb)),
            pl.BlockSpec((HID, CIN), lambda m, fb: (0, 0)),
            pl.BlockSpec((HID, CIN), lambda m, fb: (0, 0)),
        ],
        out_specs=[
            pl.BlockSpec((FB, HID), lambda m, fb: (m * NMB + fb, 0)),
            pl.BlockSpec((FB, HID), lambda m, fb: (m * NMB + fb, 0)),
        ],
        out_shape=[jax.ShapeDtypeStruct((MF, HID), jnp.float32),
                   jax.ShapeDtypeStruct((MF, HID), jnp.bfloat16)],
    )(fea, W1_0, Wp)

    s1, st1 = _gsum(z1, idx_all)
    z2 = _bnmm(s1, st1, g1_0r, be1_0r, W2_0)
    s2, st2 = _gsum(z2, idx_all)
    z3, h0 = pl.pallas_call(
        _bnmm4_body,
        grid=(GRID,),
        in_specs=[_sblk, _st, _row, _row, _full, _sblk],
        out_specs=[_sblk, _sblk],
        out_shape=[jax.ShapeDtypeStruct((MF, HID), jnp.float32)] * 2,
    )(s2, st2, g2_0r, be2_0r, Wh, p)
    s3, st3 = _gsum(z3, idx_all)
    z4 = _bnmm(s3, st3, g1_1r, be1_1r, W2_1)
    s4, st4 = _gsum(z4, idx_all)
    out = pl.pallas_call(
        _final_body,
        grid=(M, NMB),
        in_specs=[
            pl.BlockSpec((1, 8, 128), lambda m, fb: (0, 0, 0)),
            pl.BlockSpec((FB, HID), lambda m, fb: (m * NMB + fb, 0)),
            pl.BlockSpec((FB, HID), lambda m, fb: (m * NMB + fb, 0)),
            pl.BlockSpec((2 * NW, HID), lambda m, fb: (0, 0)),
            pl.BlockSpec((1, HID), lambda m, fb: (0, 0)),
            pl.BlockSpec((1, HID), lambda m, fb: (0, 0)),
        ],
        out_specs=pl.BlockSpec((1, 2 * HID, FB), lambda m, fb: (m, 1, fb)),
        out_shape=jax.ShapeDtypeStruct((M, CIN + 2 * HID, F), jnp.float32),
        input_output_aliases={0: 0},
    )(out0, h0, s4, st4, g2_1r, be2_1r)
    return out


# speedup vs baseline: 1.0901x; 1.0062x over previous
import functools

import jax
import jax.numpy as jnp
from jax import lax
from jax.experimental import pallas as pl
from jax.experimental.pallas import tpu as pltpu
from jax.experimental.pallas import tpu_sc as plsc

M, F, K = 4, 4096, 12
CIN, HID = 256, 128
MF = M * F
FB = 512
NMB = F // FB
GRID = MF // FB
NW = 32
RPW = MF // NW
SPW = RPW // 8
GW = 12 * 8
NV = HID // 16
N_TOT = float(MF)
EPS = 1e-5

_mesh = plsc.VectorSubcoreMesh(core_axis_name="c", subcore_axis_name="s")


@functools.partial(
    pl.kernel,
    out_type=[jax.ShapeDtypeStruct((MF, HID), jnp.float32),
              jax.ShapeDtypeStruct((2 * NW, HID), jnp.float32)],
    mesh=_mesh,
    scratch_types=[
        pltpu.VMEM((SPW, GW), jnp.int32),
        [pltpu.VMEM((GW, HID), jnp.float32)] * 2,
        [pltpu.VMEM((8, HID), jnp.float32)] * 2,
        [pltpu.VMEM((8, HID), jnp.float32)] * 2,
        pltpu.VMEM((2, HID), jnp.float32),
        [pltpu.SemaphoreType.DMA] * 2,
        [pltpu.SemaphoreType.DMA] * 2,
        [pltpu.SemaphoreType.DMA] * 2,
    ],
)
def _gsum(z_hbm, idx_hbm, out_hbm, st_hbm, idx_v, rows, cen, outv,
          st_v, sg, sc, so):
    wid = lax.axis_index("s") * 2 + lax.axis_index("c")
    pltpu.sync_copy(idx_hbm.at[wid], idx_v)

    zero = jnp.zeros((16,), jnp.float32)
    for v in range(NV):
        st_v[0, pl.ds(v * 16, 16)] = zero
        st_v[1, pl.ds(v * 16, 16)] = zero

    for b in range(2):
        pltpu.async_copy(z_hbm.at[idx_v.at[b]], rows[b], sg[b])
        pltpu.async_copy(z_hbm.at[pl.ds(wid * RPW + b * 8, 8)], cen[b], sc[b])

    def body(i, carry):
        for b in range(2):
            s = 2 * i + b
            pltpu.make_async_copy(z_hbm.at[idx_v.at[s]], rows[b], sg[b]).wait()
            pltpu.make_async_copy(z_hbm.at[pl.ds(wid * RPW + s * 8, 8)],
                                  cen[b], sc[b]).wait()

            @pl.when(i > 0)
            def _drain():
                pltpu.make_async_copy(
                    outv[b], out_hbm.at[pl.ds(wid * RPW + (s - 2) * 8, 8)],
                    so[b]).wait()

            for v in range(NV):
                sl = pl.ds(v * 16, 16)
                faces = []
                for r in range(8):
                    acc = cen[b][r, sl]
                    for j in range(12):
                        acc = acc + rows[b][r * 12 + j, sl]
                    outv[b][r, sl] = acc
                    faces.append(acc)
                def _tree(xs):
                    while len(xs) > 1:
                        xs = [a + c for a, c in zip(xs[::2], xs[1::2])]
                    return xs[0]
                plsc.addupdate(st_v.at[0, sl], _tree(faces))
                plsc.addupdate(st_v.at[1, sl], _tree([a * a for a in faces]))
            pltpu.async_copy(outv[b], out_hbm.at[pl.ds(wid * RPW + s * 8, 8)],
                             so[b])

            @pl.when(i < SPW // 2 - 1)
            def _prefetch():
                pltpu.async_copy(z_hbm.at[idx_v.at[s + 2]], rows[b], sg[b])
                pltpu.async_copy(z_hbm.at[pl.ds(wid * RPW + (s + 2) * 8, 8)],
                                 cen[b], sc[b])
        return carry

    lax.fori_loop(0, SPW // 2, body, 0)

    for b in range(2):
        pltpu.make_async_copy(
            outv[b], out_hbm.at[pl.ds(wid * RPW + (SPW - 2 + b) * 8, 8)],
            so[b]).wait()
    pltpu.sync_copy(st_v.at[pl.ds(0, 1)], st_hbm.at[pl.ds(wid, 1)])
    pltpu.sync_copy(st_v.at[pl.ds(1, 1)], st_hbm.at[pl.ds(NW + wid, 1)])


def _mm0_body(fea_ref, w1_ref, wp_ref, z_ref, p_ref):
    x = fea_ref[0]
    dn = (((0,), (1,)), ((), ()))
    z_ref[...] = lax.dot_general(x, w1_ref[...], dn, preferred_element_type=jnp.float32)
    p_ref[...] = lax.dot_general(x, wp_ref[...], dn,
                                 preferred_element_type=jnp.float32
                                 ).astype(jnp.bfloat16)


def _bn_act(s_ref, st_ref, g_ref, be_ref):
    st = st_ref[...]
    mean = jnp.sum(st[0:NW], axis=0, keepdims=True) * (1.0 / N_TOT)
    var = jnp.sum(st[NW:], axis=0, keepdims=True) * (1.0 / N_TOT) - mean * mean
    scale = g_ref[...] * lax.rsqrt(var + EPS)
    return jnp.maximum((s_ref[...] - mean) * scale + be_ref[...], 0.0)


def _bnmm_body(s_ref, st_ref, g_ref, be_ref, w_ref, z_ref):
    a = _bn_act(s_ref, st_ref, g_ref, be_ref)
    z_ref[...] = lax.dot_general(a, w_ref[...], (((1,), (1,)), ((), ())),
                                 preferred_element_type=jnp.float32)


def _bnmm4_body(s_ref, st_ref, g_ref, be_ref, w_ref, p_ref, z_ref, h_ref):
    h = _bn_act(s_ref, st_ref, g_ref, be_ref)
    h_ref[...] = h
    z_ref[...] = (p_ref[...].astype(jnp.float32)
                  + lax.dot_general(h, w_ref[...], (((1,), (1,)), ((), ())),
                                    preferred_element_type=jnp.float32))


def _copy_body(fea_ref, o_ref):
    o_ref[...] = fea_ref[...]


def _final_body(buf_ref, h0_ref, s_ref, st_ref, g_ref, be_ref, o_ref):
    del buf_ref
    h1 = _bn_act(s_ref, st_ref, g_ref, be_ref)
    o_ref[0] = jnp.concatenate([h0_ref[...].T, h1.T], axis=0)


_full = pl.BlockSpec((HID, HID), lambda i: (0, 0))
_row = pl.BlockSpec((1, HID), lambda i: (0, 0))
_st = pl.BlockSpec((2 * NW, HID), lambda i: (0, 0))
_sblk = pl.BlockSpec((FB, HID), lambda i: (i, 0))


def _bnmm(s, st, g, be, w):
    return pl.pallas_call(
        _bnmm_body,
        grid=(GRID,),
        in_specs=[_sblk, _st, _row, _row, _full],
        out_specs=_sblk,
        out_shape=jax.ShapeDtypeStruct((MF, HID), jnp.float32),
    )(s, st, g, be, w)


def kernel(fea, ring_n, W1_0, b1_0, g1_0, be1_0, W2_0, b2_0, g2_0, be2_0,
           W1_1, b1_1, g1_1, be1_1, W2_1, b2_1, g2_1, be2_1):
    ring = ring_n.astype(jnp.int32)
    base = (jnp.arange(M, dtype=jnp.int32) * F)[:, None, None]
    idx_all = (ring + base).reshape(NW, SPW, GW)

    g1_0r, be1_0r = g1_0.reshape(1, HID), be1_0.reshape(1, HID)
    g2_0r, be2_0r = g2_0.reshape(1, HID), be2_0.reshape(1, HID)
    g1_1r, be1_1r = g1_1.reshape(1, HID), be1_1.reshape(1, HID)
    g2_1r, be2_1r = g2_1.reshape(1, HID), be2_1.reshape(1, HID)
    Wp = W1_1[:, :CIN]
    Wh = W1_1[:, CIN:]

    out0 = pl.pallas_call(
        _copy_body,
        grid=(M, NMB),
        in_specs=[pl.BlockSpec((1, CIN, FB), lambda m, fb: (m, 0, fb))],
        out_specs=pl.BlockSpec((1, CIN, FB), lambda m, fb: (m, 0, fb)),
        out_shape=jax.ShapeDtypeStruct((M, CIN + 2 * HID, F), jnp.float32),
    )(fea)

    z1, p = pl.pallas_call(
        _mm0_body,
        grid=(M, NMB),
        in_specs=[
            pl.BlockSpec((1, CIN, FB), lambda m, fb: (m, 0, fb)),
            pl.BlockSpec((HID, CIN), lambda m, fb: (0, 0)),
            pl.BlockSpec((HID, CIN), lambda m, fb: (0, 0)),
        ],
        out_specs=[
            pl.BlockSpec((FB, HID), lambda m, fb: (m * NMB + fb, 0)),
            pl.BlockSpec((FB, HID), lambda m, fb: (m * NMB + fb, 0)),
        ],
        out_shape=[jax.ShapeDtypeStruct((MF, HID), jnp.float32),
                   jax.ShapeDtypeStruct((MF, HID), jnp.bfloat16)],
    )(fea, W1_0, Wp)

    s1, st1 = _gsum(z1, idx_all)
    z2 = _bnmm(s1, st1, g1_0r, be1_0r, W2_0)
    s2, st2 = _gsum(z2, idx_all)
    z3, h0 = pl.pallas_call(
        _bnmm4_body,
        grid=(GRID,),
        in_specs=[_sblk, _st, _row, _row, _full, _sblk],
        out_specs=[_sblk, _sblk],
        out_shape=[jax.ShapeDtypeStruct((MF, HID), jnp.float32)] * 2,
    )(s2, st2, g2_0r, be2_0r, Wh, p)
    s3, st3 = _gsum(z3, idx_all)
    z4 = _bnmm(s3, st3, g1_1r, be1_1r, W2_1)
    s4, st4 = _gsum(z4, idx_all)
    out = pl.pallas_call(
        _final_body,
        grid=(M, NMB),
        in_specs=[
            pl.BlockSpec((1, 8, 128), lambda m, fb: (0, 0, 0)),
            pl.BlockSpec((FB, HID), lambda m, fb: (m * NMB + fb, 0)),
            pl.BlockSpec((FB, HID), lambda m, fb: (m * NMB + fb, 0)),
            pl.BlockSpec((2 * NW, HID), lambda m, fb: (0, 0)),
            pl.BlockSpec((1, HID), lambda m, fb: (0, 0)),
            pl.BlockSpec((1, HID), lambda m, fb: (0, 0)),
        ],
        out_specs=pl.BlockSpec((1, 2 * HID, FB), lambda m, fb: (m, 1, fb)),
        out_shape=jax.ShapeDtypeStruct((M, CIN + 2 * HID, F), jnp.float32),
        input_output_aliases={0: 0},
    )(out0, h0, s4, st4, g2_1r, be2_1r)
    return out


# speedup vs baseline: 1.2148x; 1.1145x over previous
import functools

import jax
import jax.numpy as jnp
from jax import lax
from jax.experimental import pallas as pl
from jax.experimental.pallas import tpu as pltpu
from jax.experimental.pallas import tpu_sc as plsc

M, F, K = 4, 4096, 12
CIN, HID = 256, 128
MF = M * F
FB = 2048
NMB = F // FB
GRID = MF // FB
NW = 32
RPW = MF // NW
SPW = RPW // 8
GW = 12 * 8
NV = HID // 16
N_TOT = float(MF)
EPS = 1e-5

_mesh = plsc.VectorSubcoreMesh(core_axis_name="c", subcore_axis_name="s")


@functools.partial(
    pl.kernel,
    out_type=[jax.ShapeDtypeStruct((MF, HID), jnp.float32),
              jax.ShapeDtypeStruct((2 * NW, HID), jnp.float32)],
    mesh=_mesh,
    scratch_types=[
        pltpu.VMEM((SPW, GW), jnp.int32),
        [pltpu.VMEM((GW, HID), jnp.float32)] * 2,
        [pltpu.VMEM((8, HID), jnp.float32)] * 2,
        [pltpu.VMEM((8, HID), jnp.float32)] * 2,
        pltpu.VMEM((2, HID), jnp.float32),
        [pltpu.SemaphoreType.DMA] * 2,
        [pltpu.SemaphoreType.DMA] * 2,
        [pltpu.SemaphoreType.DMA] * 2,
    ],
)
def _gsum(z_hbm, idx_hbm, out_hbm, st_hbm, idx_v, rows, cen, outv,
          st_v, sg, sc, so):
    wid = lax.axis_index("s") * 2 + lax.axis_index("c")
    pltpu.sync_copy(idx_hbm.at[wid], idx_v)

    zero = jnp.zeros((16,), jnp.float32)
    for v in range(NV):
        st_v[0, pl.ds(v * 16, 16)] = zero
        st_v[1, pl.ds(v * 16, 16)] = zero

    for b in range(2):
        pltpu.async_copy(z_hbm.at[idx_v.at[b]], rows[b], sg[b])
        pltpu.async_copy(z_hbm.at[pl.ds(wid * RPW + b * 8, 8)], cen[b], sc[b])

    def body(i, carry):
        for b in range(2):
            s = 2 * i + b
            pltpu.make_async_copy(z_hbm.at[idx_v.at[s]], rows[b], sg[b]).wait()
            pltpu.make_async_copy(z_hbm.at[pl.ds(wid * RPW + s * 8, 8)],
                                  cen[b], sc[b]).wait()

            @pl.when(i > 0)
            def _drain():
                pltpu.make_async_copy(
                    outv[b], out_hbm.at[pl.ds(wid * RPW + (s - 2) * 8, 8)],
                    so[b]).wait()

            for v in range(NV):
                sl = pl.ds(v * 16, 16)
                faces = []
                for r in range(8):
                    acc = cen[b][r, sl]
                    for j in range(12):
                        acc = acc + rows[b][r * 12 + j, sl]
                    outv[b][r, sl] = acc
                    faces.append(acc)
                def _tree(xs):
                    while len(xs) > 1:
                        xs = [a + c for a, c in zip(xs[::2], xs[1::2])]
                    return xs[0]
                plsc.addupdate(st_v.at[0, sl], _tree(faces))
                plsc.addupdate(st_v.at[1, sl], _tree([a * a for a in faces]))
            pltpu.async_copy(outv[b], out_hbm.at[pl.ds(wid * RPW + s * 8, 8)],
                             so[b])

            @pl.when(i < SPW // 2 - 1)
            def _prefetch():
                pltpu.async_copy(z_hbm.at[idx_v.at[s + 2]], rows[b], sg[b])
                pltpu.async_copy(z_hbm.at[pl.ds(wid * RPW + (s + 2) * 8, 8)],
                                 cen[b], sc[b])
        return carry

    lax.fori_loop(0, SPW // 2, body, 0)

    for b in range(2):
        pltpu.make_async_copy(
            outv[b], out_hbm.at[pl.ds(wid * RPW + (SPW - 2 + b) * 8, 8)],
            so[b]).wait()
    pltpu.sync_copy(st_v.at[pl.ds(0, 1)], st_hbm.at[pl.ds(wid, 1)])
    pltpu.sync_copy(st_v.at[pl.ds(1, 1)], st_hbm.at[pl.ds(NW + wid, 1)])


def _mm0_body(fea_ref, w1_ref, wp_ref, z_ref, p_ref):
    x = fea_ref[0]
    dn = (((0,), (1,)), ((), ()))
    z_ref[...] = lax.dot_general(x, w1_ref[...], dn, preferred_element_type=jnp.float32)
    p_ref[...] = lax.dot_general(x, wp_ref[...], dn,
                                 preferred_element_type=jnp.float32
                                 ).astype(jnp.bfloat16)


def _bn_act(s_ref, st_ref, g_ref, be_ref):
    st = st_ref[...]
    mean = jnp.sum(st[0:NW], axis=0, keepdims=True) * (1.0 / N_TOT)
    var = jnp.sum(st[NW:], axis=0, keepdims=True) * (1.0 / N_TOT) - mean * mean
    scale = g_ref[...] * lax.rsqrt(var + EPS)
    return jnp.maximum((s_ref[...] - mean) * scale + be_ref[...], 0.0)


def _bnmm_body(s_ref, st_ref, g_ref, be_ref, w_ref, z_ref):
    a = _bn_act(s_ref, st_ref, g_ref, be_ref)
    z_ref[...] = lax.dot_general(a, w_ref[...], (((1,), (1,)), ((), ())),
                                 preferred_element_type=jnp.float32)


def _bnmm4_body(s_ref, st_ref, g_ref, be_ref, w_ref, p_ref, z_ref, h_ref):
    h = _bn_act(s_ref, st_ref, g_ref, be_ref)
    h_ref[...] = h
    z_ref[...] = (p_ref[...].astype(jnp.float32)
                  + lax.dot_general(h, w_ref[...], (((1,), (1,)), ((), ())),
                                    preferred_element_type=jnp.float32))


def _copy_body(fea_ref, o_ref):
    o_ref[...] = fea_ref[...]


def _final_body(buf_ref, h0_ref, s_ref, st_ref, g_ref, be_ref, o_ref):
    del buf_ref
    h1 = _bn_act(s_ref, st_ref, g_ref, be_ref)
    o_ref[0] = jnp.concatenate([h0_ref[...].T, h1.T], axis=0)


_full = pl.BlockSpec((HID, HID), lambda i: (0, 0))
_row = pl.BlockSpec((1, HID), lambda i: (0, 0))
_st = pl.BlockSpec((2 * NW, HID), lambda i: (0, 0))
_sblk = pl.BlockSpec((FB, HID), lambda i: (i, 0))


def _bnmm(s, st, g, be, w):
    return pl.pallas_call(
        _bnmm_body,
        grid=(GRID,),
        in_specs=[_sblk, _st, _row, _row, _full],
        out_specs=_sblk,
        out_shape=jax.ShapeDtypeStruct((MF, HID), jnp.float32),
    )(s, st, g, be, w)


def kernel(fea, ring_n, W1_0, b1_0, g1_0, be1_0, W2_0, b2_0, g2_0, be2_0,
           W1_1, b1_1, g1_1, be1_1, W2_1, b2_1, g2_1, be2_1):
    ring = ring_n.astype(jnp.int32)
    base = (jnp.arange(M, dtype=jnp.int32) * F)[:, None, None]
    idx_all = (ring + base).reshape(NW, SPW, GW)

    g1_0r, be1_0r = g1_0.reshape(1, HID), be1_0.reshape(1, HID)
    g2_0r, be2_0r = g2_0.reshape(1, HID), be2_0.reshape(1, HID)
    g1_1r, be1_1r = g1_1.reshape(1, HID), be1_1.reshape(1, HID)
    g2_1r, be2_1r = g2_1.reshape(1, HID), be2_1.reshape(1, HID)
    Wp = W1_1[:, :CIN]
    Wh = W1_1[:, CIN:]

    out0 = pl.pallas_call(
        _copy_body,
        grid=(M, NMB),
        in_specs=[pl.BlockSpec((1, CIN, FB), lambda m, fb: (m, 0, fb))],
        out_specs=pl.BlockSpec((1, CIN, FB), lambda m, fb: (m, 0, fb)),
        out_shape=jax.ShapeDtypeStruct((M, CIN + 2 * HID, F), jnp.float32),
    )(fea)

    z1, p = pl.pallas_call(
        _mm0_body,
        grid=(M, NMB),
        in_specs=[
            pl.BlockSpec((1, CIN, FB), lambda m, fb: (m, 0, fb)),
            pl.BlockSpec((HID, CIN), lambda m, fb: (0, 0)),
            pl.BlockSpec((HID, CIN), lambda m, fb: (0, 0)),
        ],
        out_specs=[
            pl.BlockSpec((FB, HID), lambda m, fb: (m * NMB + fb, 0)),
            pl.BlockSpec((FB, HID), lambda m, fb: (m * NMB + fb, 0)),
        ],
        out_shape=[jax.ShapeDtypeStruct((MF, HID), jnp.float32),
                   jax.ShapeDtypeStruct((MF, HID), jnp.bfloat16)],
    )(fea, W1_0, Wp)

    s1, st1 = _gsum(z1, idx_all)
    z2 = _bnmm(s1, st1, g1_0r, be1_0r, W2_0)
    s2, st2 = _gsum(z2, idx_all)
    z3, h0 = pl.pallas_call(
        _bnmm4_body,
        grid=(GRID,),
        in_specs=[_sblk, _st, _row, _row, _full, _sblk],
        out_specs=[_sblk, _sblk],
        out_shape=[jax.ShapeDtypeStruct((MF, HID), jnp.float32)] * 2,
    )(s2, st2, g2_0r, be2_0r, Wh, p)
    s3, st3 = _gsum(z3, idx_all)
    z4 = _bnmm(s3, st3, g1_1r, be1_1r, W2_1)
    s4, st4 = _gsum(z4, idx_all)
    out = pl.pallas_call(
        _final_body,
        grid=(M, NMB),
        in_specs=[
            pl.BlockSpec((1, 8, 128), lambda m, fb: (0, 0, 0)),
            pl.BlockSpec((FB, HID), lambda m, fb: (m * NMB + fb, 0)),
            pl.BlockSpec((FB, HID), lambda m, fb: (m * NMB + fb, 0)),
            pl.BlockSpec((2 * NW, HID), lambda m, fb: (0, 0)),
            pl.BlockSpec((1, HID), lambda m, fb: (0, 0)),
            pl.BlockSpec((1, HID), lambda m, fb: (0, 0)),
        ],
        out_specs=pl.BlockSpec((1, 2 * HID, FB), lambda m, fb: (m, 1, fb)),
        out_shape=jax.ShapeDtypeStruct((M, CIN + 2 * HID, F), jnp.float32),
        input_output_aliases={0: 0},
    )(out0, h0, s4, st4, g2_1r, be2_1r)
    return out


# speedup vs baseline: 1.2348x; 1.0164x over previous
import functools

import jax
import jax.numpy as jnp
from jax import lax
from jax.experimental import pallas as pl
from jax.experimental.pallas import tpu as pltpu
from jax.experimental.pallas import tpu_sc as plsc

M, F, K = 4, 4096, 12
CIN, HID = 256, 128
MF = M * F
FB = 4096
NMB = F // FB
GRID = MF // FB
NW = 32
RPW = MF // NW
SPW = RPW // 8
GW = 12 * 8
NV = HID // 16
N_TOT = float(MF)
EPS = 1e-5

_mesh = plsc.VectorSubcoreMesh(core_axis_name="c", subcore_axis_name="s")


@functools.partial(
    pl.kernel,
    out_type=[jax.ShapeDtypeStruct((MF, HID), jnp.float32),
              jax.ShapeDtypeStruct((2 * NW, HID), jnp.float32)],
    mesh=_mesh,
    scratch_types=[
        pltpu.VMEM((SPW, GW), jnp.int32),
        [pltpu.VMEM((GW, HID), jnp.float32)] * 2,
        [pltpu.VMEM((8, HID), jnp.float32)] * 2,
        [pltpu.VMEM((8, HID), jnp.float32)] * 2,
        pltpu.VMEM((2, HID), jnp.float32),
        [pltpu.SemaphoreType.DMA] * 2,
        [pltpu.SemaphoreType.DMA] * 2,
        [pltpu.SemaphoreType.DMA] * 2,
    ],
)
def _gsum(z_hbm, idx_hbm, out_hbm, st_hbm, idx_v, rows, cen, outv,
          st_v, sg, sc, so):
    wid = lax.axis_index("s") * 2 + lax.axis_index("c")
    pltpu.sync_copy(idx_hbm.at[wid], idx_v)

    zero = jnp.zeros((16,), jnp.float32)
    for v in range(NV):
        st_v[0, pl.ds(v * 16, 16)] = zero
        st_v[1, pl.ds(v * 16, 16)] = zero

    for b in range(2):
        pltpu.async_copy(z_hbm.at[idx_v.at[b]], rows[b], sg[b])
        pltpu.async_copy(z_hbm.at[pl.ds(wid * RPW + b * 8, 8)], cen[b], sc[b])

    def body(i, carry):
        for b in range(2):
            s = 2 * i + b
            pltpu.make_async_copy(z_hbm.at[idx_v.at[s]], rows[b], sg[b]).wait()
            pltpu.make_async_copy(z_hbm.at[pl.ds(wid * RPW + s * 8, 8)],
                                  cen[b], sc[b]).wait()

            @pl.when(i > 0)
            def _drain():
                pltpu.make_async_copy(
                    outv[b], out_hbm.at[pl.ds(wid * RPW + (s - 2) * 8, 8)],
                    so[b]).wait()

            for v in range(NV):
                sl = pl.ds(v * 16, 16)
                faces = []
                for r in range(8):
                    acc = cen[b][r, sl]
                    for j in range(12):
                        acc = acc + rows[b][r * 12 + j, sl]
                    outv[b][r, sl] = acc
                    faces.append(acc)
                def _tree(xs):
                    while len(xs) > 1:
                        xs = [a + c for a, c in zip(xs[::2], xs[1::2])]
                    return xs[0]
                plsc.addupdate(st_v.at[0, sl], _tree(faces))
                plsc.addupdate(st_v.at[1, sl], _tree([a * a for a in faces]))
            pltpu.async_copy(outv[b], out_hbm.at[pl.ds(wid * RPW + s * 8, 8)],
                             so[b])

            @pl.when(i < SPW // 2 - 1)
            def _prefetch():
                pltpu.async_copy(z_hbm.at[idx_v.at[s + 2]], rows[b], sg[b])
                pltpu.async_copy(z_hbm.at[pl.ds(wid * RPW + (s + 2) * 8, 8)],
                                 cen[b], sc[b])
        return carry

    lax.fori_loop(0, SPW // 2, body, 0)

    for b in range(2):
        pltpu.make_async_copy(
            outv[b], out_hbm.at[pl.ds(wid * RPW + (SPW - 2 + b) * 8, 8)],
            so[b]).wait()
    pltpu.sync_copy(st_v.at[pl.ds(0, 1)], st_hbm.at[pl.ds(wid, 1)])
    pltpu.sync_copy(st_v.at[pl.ds(1, 1)], st_hbm.at[pl.ds(NW + wid, 1)])


def _mm0_body(fea_ref, w1_ref, wp_ref, z_ref, p_ref):
    x = fea_ref[0]
    dn = (((0,), (1,)), ((), ()))
    z_ref[...] = lax.dot_general(x, w1_ref[...], dn, preferred_element_type=jnp.float32)
    p_ref[...] = lax.dot_general(x, wp_ref[...], dn,
                                 preferred_element_type=jnp.float32
                                 ).astype(jnp.bfloat16)


def _bn_act(s_ref, st_ref, g_ref, be_ref):
    st = st_ref[...]
    mean = jnp.sum(st[0:NW], axis=0, keepdims=True) * (1.0 / N_TOT)
    var = jnp.sum(st[NW:], axis=0, keepdims=True) * (1.0 / N_TOT) - mean * mean
    scale = g_ref[...] * lax.rsqrt(var + EPS)
    return jnp.maximum((s_ref[...] - mean) * scale + be_ref[...], 0.0)


def _bnmm_body(s_ref, st_ref, g_ref, be_ref, w_ref, z_ref):
    a = _bn_act(s_ref, st_ref, g_ref, be_ref)
    z_ref[...] = lax.dot_general(a, w_ref[...], (((1,), (1,)), ((), ())),
                                 preferred_element_type=jnp.float32)


def _bnmm4_body(s_ref, st_ref, g_ref, be_ref, w_ref, p_ref, z_ref, h_ref):
    h = _bn_act(s_ref, st_ref, g_ref, be_ref)
    h_ref[...] = h
    z_ref[...] = (p_ref[...].astype(jnp.float32)
                  + lax.dot_general(h, w_ref[...], (((1,), (1,)), ((), ())),
                                    preferred_element_type=jnp.float32))


def _copy_body(fea_ref, o_ref):
    o_ref[...] = fea_ref[...]


def _final_body(buf_ref, h0_ref, s_ref, st_ref, g_ref, be_ref, o_ref):
    del buf_ref
    h1 = _bn_act(s_ref, st_ref, g_ref, be_ref)
    o_ref[0] = jnp.concatenate([h0_ref[...].T, h1.T], axis=0)


_full = pl.BlockSpec((HID, HID), lambda i: (0, 0))
_row = pl.BlockSpec((1, HID), lambda i: (0, 0))
_st = pl.BlockSpec((2 * NW, HID), lambda i: (0, 0))
_sblk = pl.BlockSpec((FB, HID), lambda i: (i, 0))


def _bnmm(s, st, g, be, w):
    return pl.pallas_call(
        _bnmm_body,
        grid=(GRID,),
        in_specs=[_sblk, _st, _row, _row, _full],
        out_specs=_sblk,
        out_shape=jax.ShapeDtypeStruct((MF, HID), jnp.float32),
    )(s, st, g, be, w)


def kernel(fea, ring_n, W1_0, b1_0, g1_0, be1_0, W2_0, b2_0, g2_0, be2_0,
           W1_1, b1_1, g1_1, be1_1, W2_1, b2_1, g2_1, be2_1):
    ring = ring_n.astype(jnp.int32)
    base = (jnp.arange(M, dtype=jnp.int32) * F)[:, None, None]
    idx_all = (ring + base).reshape(NW, SPW, GW)

    g1_0r, be1_0r = g1_0.reshape(1, HID), be1_0.reshape(1, HID)
    g2_0r, be2_0r = g2_0.reshape(1, HID), be2_0.reshape(1, HID)
    g1_1r, be1_1r = g1_1.reshape(1, HID), be1_1.reshape(1, HID)
    g2_1r, be2_1r = g2_1.reshape(1, HID), be2_1.reshape(1, HID)
    Wp = W1_1[:, :CIN]
    Wh = W1_1[:, CIN:]

    out0 = pl.pallas_call(
        _copy_body,
        grid=(M, NMB),
        in_specs=[pl.BlockSpec((1, CIN, FB), lambda m, fb: (m, 0, fb))],
        out_specs=pl.BlockSpec((1, CIN, FB), lambda m, fb: (m, 0, fb)),
        out_shape=jax.ShapeDtypeStruct((M, CIN + 2 * HID, F), jnp.float32),
    )(fea)

    z1, p = pl.pallas_call(
        _mm0_body,
        grid=(M, NMB),
        in_specs=[
            pl.BlockSpec((1, CIN, FB), lambda m, fb: (m, 0, fb)),
            pl.BlockSpec((HID, CIN), lambda m, fb: (0, 0)),
            pl.BlockSpec((HID, CIN), lambda m, fb: (0, 0)),
        ],
        out_specs=[
            pl.BlockSpec((FB, HID), lambda m, fb: (m * NMB + fb, 0)),
            pl.BlockSpec((FB, HID), lambda m, fb: (m * NMB + fb, 0)),
        ],
        out_shape=[jax.ShapeDtypeStruct((MF, HID), jnp.float32),
                   jax.ShapeDtypeStruct((MF, HID), jnp.bfloat16)],
    )(fea, W1_0, Wp)

    s1, st1 = _gsum(z1, idx_all)
    z2 = _bnmm(s1, st1, g1_0r, be1_0r, W2_0)
    s2, st2 = _gsum(z2, idx_all)
    z3, h0 = pl.pallas_call(
        _bnmm4_body,
        grid=(GRID,),
        in_specs=[_sblk, _st, _row, _row, _full, _sblk],
        out_specs=[_sblk, _sblk],
        out_shape=[jax.ShapeDtypeStruct((MF, HID), jnp.float32)] * 2,
    )(s2, st2, g2_0r, be2_0r, Wh, p)
    s3, st3 = _gsum(z3, idx_all)
    z4 = _bnmm(s3, st3, g1_1r, be1_1r, W2_1)
    s4, st4 = _gsum(z4, idx_all)
    out = pl.pallas_call(
        _final_body,
        grid=(M, NMB),
        in_specs=[
            pl.BlockSpec((1, 8, 128), lambda m, fb: (0, 0, 0)),
            pl.BlockSpec((FB, HID), lambda m, fb: (m * NMB + fb, 0)),
            pl.BlockSpec((FB, HID), lambda m, fb: (m * NMB + fb, 0)),
            pl.BlockSpec((2 * NW, HID), lambda m, fb: (0, 0)),
            pl.BlockSpec((1, HID), lambda m, fb: (0, 0)),
            pl.BlockSpec((1, HID), lambda m, fb: (0, 0)),
        ],
        out_specs=pl.BlockSpec((1, 2 * HID, FB), lambda m, fb: (m, 1, fb)),
        out_shape=jax.ShapeDtypeStruct((M, CIN + 2 * HID, F), jnp.float32),
        input_output_aliases={0: 0},
    )(out0, h0, s4, st4, g2_1r, be2_1r)
    return out
